# baseline (speedup 1.0000x reference)
import math

import jax
import jax.numpy as jnp
from jax import lax
from jax.experimental import pallas as pl
from jax.experimental.pallas import tpu as pltpu
from jax.experimental.pallas import tpu_sc as plsc

D_MODEL = 1024
SCALE = math.sqrt(D_MODEL)
LANES = 16
VPR = D_MODEL // LANES

NC = 2
NS = 16
NW = NC * NS

B_TOTAL = 4 * 4096
B_PER_W = B_TOTAL // NW
CHUNK = 8
NCHUNK = B_PER_W // CHUNK
NBUF = 14
LOOKAHEAD = 10
UNROLL = 8


def _emb_kernel(x_hbm, table_hbm, out_hbm, idx_v, ring, gsem, ssem):
  wid = lax.axis_index("s") * NC + lax.axis_index("c")
  base = wid * B_PER_W

  pltpu.sync_copy(x_hbm.at[pl.ds(base, B_PER_W)], idx_v)

  def gather_copy(c, s):
    return pltpu.make_async_copy(
        table_hbm.at[idx_v.at[pl.ds(c * CHUNK, CHUNK)]], ring.at[s],
        gsem.at[s])

  b_idx = base // (B_TOTAL // 4)
  b_off = lax.rem(base, B_TOTAL // 4)

  def store_copy(c, s):
    return pltpu.make_async_copy(
        ring.at[s], out_hbm.at[b_idx, pl.ds(b_off + c * CHUNK, CHUNK)],
        ssem.at[s])

  def scale_chunk(s):
    @plsc.parallel_loop(0, CHUNK * VPR, step=1, unroll=UNROLL)
    def _body(i):
      r = lax.shift_right_logical(i, 6)
      off = lax.shift_left(lax.bitwise_and(i, VPR - 1), 4)
      sl = pl.ds(pl.multiple_of(off, LANES), LANES)
      ring[s, r, sl] = ring[s, r, sl] * SCALE

  for c in range(LOOKAHEAD):
    gather_copy(c, c).start()

  def step(c, carry):
    g = c + LOOKAHEAD

    @pl.when(g < NCHUNK)
    def _():
      sg = lax.rem(g, NBUF)

      @pl.when(g >= NBUF)
      def _():
        store_copy(g - NBUF, sg).wait()

      gather_copy(g, sg).start()

    s = lax.rem(c, NBUF)
    gather_copy(c, s).wait()
    scale_chunk(s)
    store_copy(c, s).start()
    return carry

  lax.fori_loop(0, NCHUNK, step, 0)

  for c in range(NCHUNK - NBUF, NCHUNK):
    store_copy(c, c % NBUF).wait()


@jax.jit
def kernel(x, table):
  idx = x.reshape(-1).astype(jnp.int32)
  mesh = plsc.VectorSubcoreMesh(
      core_axis_name="c", subcore_axis_name="s", num_cores=NC,
      num_subcores=NS)
  run = pl.kernel(
      _emb_kernel,
      out_type=jax.ShapeDtypeStruct((4, B_TOTAL // 4, D_MODEL), jnp.float32),
      mesh=mesh,
      scratch_types=[
          pltpu.VMEM((B_PER_W,), jnp.int32),
          pltpu.VMEM((NBUF, CHUNK, D_MODEL), jnp.float32),
          pltpu.SemaphoreType.DMA((NBUF,)),
          pltpu.SemaphoreType.DMA((NBUF,)),
      ],
  )
  return run(idx, table)


# speedup vs baseline: 1.0362x; 1.0362x over previous
import math

import jax
import jax.numpy as jnp
from jax import lax
from jax.experimental import pallas as pl
from jax.experimental.pallas import tpu as pltpu
from jax.experimental.pallas import tpu_sc as plsc

D_MODEL = 1024
SCALE = math.sqrt(D_MODEL)
LANES = 16
VPR = D_MODEL // LANES

NC = 2
NS = 16
NW = NC * NS

B_TOTAL = 4 * 4096
B_PER_W = B_TOTAL // NW
CHUNK = 16
NCHUNK = B_PER_W // CHUNK
NBUF = 7
LOOKAHEAD = 6
UNROLL = 8


def _emb_kernel(x_hbm, table_hbm, out_hbm, idx_v, ring, gsem, ssem):
  wid = lax.axis_index("s") * NC + lax.axis_index("c")
  base = wid * B_PER_W

  pltpu.sync_copy(x_hbm.at[pl.ds(base, B_PER_W)], idx_v)

  def gather_copy(c, s):
    return pltpu.make_async_copy(
        table_hbm.at[idx_v.at[pl.ds(c * CHUNK, CHUNK)]], ring.at[s],
        gsem.at[s])

  b_idx = base // (B_TOTAL // 4)
  b_off = lax.rem(base, B_TOTAL // 4)

  def store_copy(c, s):
    return pltpu.make_async_copy(
        ring.at[s], out_hbm.at[b_idx, pl.ds(b_off + c * CHUNK, CHUNK)],
        ssem.at[s])

  def scale_chunk(s):
    @plsc.parallel_loop(0, CHUNK * VPR, step=1, unroll=UNROLL)
    def _body(i):
      r = lax.shift_right_logical(i, 6)
      off = lax.shift_left(lax.bitwise_and(i, VPR - 1), 4)
      sl = pl.ds(pl.multiple_of(off, LANES), LANES)
      ring[s, r, sl] = ring[s, r, sl] * SCALE

  for c in range(LOOKAHEAD):
    gather_copy(c, c).start()

  def step(c, carry):
    g = c + LOOKAHEAD

    @pl.when(g < NCHUNK)
    def _():
      sg = lax.rem(g, NBUF)

      @pl.when(g >= NBUF)
      def _():
        store_copy(g - NBUF, sg).wait()

      gather_copy(g, sg).start()

    s = lax.rem(c, NBUF)
    gather_copy(c, s).wait()
    scale_chunk(s)
    store_copy(c, s).start()
    return carry

  lax.fori_loop(0, NCHUNK, step, 0)

  for c in range(NCHUNK - NBUF, NCHUNK):
    store_copy(c, c % NBUF).wait()


@jax.jit
def kernel(x, table):
  idx = x.reshape(-1).astype(jnp.int32)
  mesh = plsc.VectorSubcoreMesh(
      core_axis_name="c", subcore_axis_name="s", num_cores=NC,
      num_subcores=NS)
  run = pl.kernel(
      _emb_kernel,
      out_type=jax.ShapeDtypeStruct((4, B_TOTAL // 4, D_MODEL), jnp.float32),
      mesh=mesh,
      scratch_types=[
          pltpu.VMEM((B_PER_W,), jnp.int32),
          pltpu.VMEM((NBUF, CHUNK, D_MODEL), jnp.float32),
          pltpu.SemaphoreType.DMA((NBUF,)),
          pltpu.SemaphoreType.DMA((NBUF,)),
      ],
  )
  return run(idx, table)


# speedup vs baseline: 1.0371x; 1.0009x over previous
import math

import jax
import jax.numpy as jnp
from jax import lax
from jax.experimental import pallas as pl
from jax.experimental.pallas import tpu as pltpu
from jax.experimental.pallas import tpu_sc as plsc

D_MODEL = 1024
SCALE = math.sqrt(D_MODEL)
LANES = 16
VPR = D_MODEL // LANES

NC = 2
NS = 16
NW = NC * NS

B_TOTAL = 4 * 4096
B_PER_W = B_TOTAL // NW
CHUNK = 16
NCHUNK = B_PER_W // CHUNK
NBUF = 7
LOOKAHEAD = 6
UNROLL = 8


def _emb_kernel(x_hbm, table_hbm, out_hbm, idx_v, ring, gsem, ssem):
  wid = lax.axis_index("s") * NC + lax.axis_index("c")
  base = wid * B_PER_W

  x_row = base // (B_TOTAL // 4)
  x_off = lax.rem(base, B_TOTAL // 4)
  pltpu.sync_copy(x_hbm.at[x_row, pl.ds(x_off, B_PER_W)], idx_v)

  def gather_copy(c, s):
    return pltpu.make_async_copy(
        table_hbm.at[idx_v.at[pl.ds(c * CHUNK, CHUNK)]], ring.at[s],
        gsem.at[s])

  b_idx = base // (B_TOTAL // 4)
  b_off = lax.rem(base, B_TOTAL // 4)

  def store_copy(c, s):
    return pltpu.make_async_copy(
        ring.at[s], out_hbm.at[b_idx, pl.ds(b_off + c * CHUNK, CHUNK)],
        ssem.at[s])

  def scale_chunk(s):
    @plsc.parallel_loop(0, CHUNK * VPR, step=1, unroll=UNROLL)
    def _body(i):
      r = lax.shift_right_logical(i, 6)
      off = lax.shift_left(lax.bitwise_and(i, VPR - 1), 4)
      sl = pl.ds(pl.multiple_of(off, LANES), LANES)
      ring[s, r, sl] = ring[s, r, sl] * SCALE

  for c in range(LOOKAHEAD):
    gather_copy(c, c).start()

  def step(c, carry):
    g = c + LOOKAHEAD

    @pl.when(g < NCHUNK)
    def _():
      sg = lax.rem(g, NBUF)

      @pl.when(g >= NBUF)
      def _():
        store_copy(g - NBUF, sg).wait()

      gather_copy(g, sg).start()

    s = lax.rem(c, NBUF)
    gather_copy(c, s).wait()
    scale_chunk(s)
    store_copy(c, s).start()
    return carry

  lax.fori_loop(0, NCHUNK, step, 0)

  for c in range(NCHUNK - NBUF, NCHUNK):
    store_copy(c, c % NBUF).wait()


@jax.jit
def kernel(x, table):
  idx = x.astype(jnp.int32)
  mesh = plsc.VectorSubcoreMesh(
      core_axis_name="c", subcore_axis_name="s", num_cores=NC,
      num_subcores=NS)
  run = pl.kernel(
      _emb_kernel,
      out_type=jax.ShapeDtypeStruct((4, B_TOTAL // 4, D_MODEL), jnp.float32),
      mesh=mesh,
      scratch_types=[
          pltpu.VMEM((B_PER_W,), jnp.int32),
          pltpu.VMEM((NBUF, CHUNK, D_MODEL), jnp.float32),
          pltpu.SemaphoreType.DMA((NBUF,)),
          pltpu.SemaphoreType.DMA((NBUF,)),
      ],
  )
  return run(idx, table)
